# baseline (speedup 1.0000x reference)
import functools

import jax
import jax.numpy as jnp
from jax import lax
from jax.experimental import pallas as pl
from jax.experimental.pallas import tpu as pltpu
from jax.experimental.pallas import tpu_sc as plsc

VOCAB = 1000
BATCH = 1024
SEQ = 50
SEQPAD = 56
D = VOCAB
DPAD = 1024
NUM_CORES = 2
NUM_SUBCORES = 16
NW = NUM_CORES * NUM_SUBCORES
BPW = BATCH // NW
NSLICE = D // 16
TAILOFF = D - 16


def _make_sc_gather():
    mesh = plsc.VectorSubcoreMesh(core_axis_name="c", subcore_axis_name="s")

    @functools.partial(
        pl.kernel,
        mesh=mesh,
        out_type=jax.ShapeDtypeStruct((BATCH, SEQ, D), jnp.float32),
        scratch_types=[
            pltpu.VMEM((BPW * SEQPAD,), jnp.int32),
            pltpu.VMEM((2, 32, DPAD), jnp.float32),
            pltpu.VMEM((SEQ, D), jnp.float32),
            pltpu.SemaphoreType.DMA,
            pltpu.SemaphoreType.DMA,
        ],
    )
    def k(table_hbm, idx_hbm, out_hbm, idx_v, gbuf, wbuf, gsem0, gsem1):
        cid = lax.axis_index("c")
        sid = lax.axis_index("s")
        wid = sid * NUM_CORES + cid
        pltpu.sync_copy(idx_hbm.at[pl.ds(wid * BPW * SEQPAD, BPW * SEQPAD)],
                        idx_v)
        gsems = (gsem0, gsem1)

        def start_a(j):
            pltpu.async_copy(
                table_hbm.at[idx_v.at[pl.ds(j * SEQPAD, 32)]],
                gbuf.at[0],
                gsems[0],
            )

        def wait_a(j):
            pltpu.make_async_copy(
                table_hbm.at[idx_v.at[pl.ds(j * SEQPAD, 32)]],
                gbuf.at[0],
                gsems[0],
            ).wait()

        def start_b(j):
            pltpu.async_copy(
                table_hbm.at[idx_v.at[pl.ds(j * SEQPAD + 32, 24)]],
                gbuf.at[1, pl.ds(0, 24)],
                gsems[1],
            )

        def wait_b(j):
            pltpu.make_async_copy(
                table_hbm.at[idx_v.at[pl.ds(j * SEQPAD + 32, 24)]],
                gbuf.at[1, pl.ds(0, 24)],
                gsems[1],
            ).wait()

        def repack(base, nrows, srcbuf):
            def row(r, carry):
                for kk in range(NSLICE):
                    wbuf[base + r, pl.ds(kk * 16, 16)] = srcbuf[
                        r, pl.ds(kk * 16, 16)
                    ]
                wbuf[base + r, pl.ds(TAILOFF, 16)] = srcbuf[
                    r, pl.ds(TAILOFF, 16)
                ]
                return carry

            lax.fori_loop(0, nrows, row, 0)

        start_a(0)
        start_b(0)

        def body(j, carry):
            bi = wid * BPW + j
            wait_a(j)
            repack(0, 32, gbuf.at[0])

            @pl.when(j + 1 < BPW)
            def _():
                start_a(j + 1)

            wait_b(j)
            repack(32, SEQ - 32, gbuf.at[1])

            @pl.when(j + 1 < BPW)
            def _():
                start_b(j + 1)

            pltpu.sync_copy(wbuf, out_hbm.at[bi])
            return carry

        lax.fori_loop(0, BPW, body, 0)

    return k


_sc_gather = _make_sc_gather()


def kernel(x, table):
    xpad = jnp.pad(x.astype(jnp.int32), ((0, 0), (0, SEQPAD - SEQ)))
    table_pad = jnp.pad(table, ((0, 0), (0, DPAD - D)))
    return _sc_gather(table_pad, xpad.reshape(-1))


# speedup vs baseline: 1.2375x; 1.2375x over previous
import functools

import jax
import jax.numpy as jnp
from jax import lax
from jax.experimental import pallas as pl
from jax.experimental.pallas import tpu as pltpu
from jax.experimental.pallas import tpu_sc as plsc

VOCAB = 1000
BATCH = 1024
SEQ = 50
N = BATCH * SEQ
D = VOCAB
NUM_CORES = 2
NUM_SUBCORES = 16
NW = NUM_CORES * NUM_SUBCORES
PER_W = N // NW
CHUNK = 64
NCHUNK = PER_W // CHUNK


def _make_sc_gather():
    mesh = plsc.VectorSubcoreMesh(core_axis_name="c", subcore_axis_name="s")

    @functools.partial(
        pl.kernel,
        mesh=mesh,
        compiler_params=pltpu.CompilerParams(use_tc_tiling_on_sc=False),
        out_type=jax.ShapeDtypeStruct((N, D), jnp.float32),
        scratch_types=[
            pltpu.VMEM((PER_W,), jnp.int32),
            pltpu.VMEM((2, CHUNK, D), jnp.float32),
            pltpu.SemaphoreType.DMA,
            pltpu.SemaphoreType.DMA,
        ],
    )
    def k(table_hbm, idx_hbm, out_hbm, idx_v, rows_v, sem0, sem1):
        cid = lax.axis_index("c")
        sid = lax.axis_index("s")
        wid = sid * NUM_CORES + cid
        base = wid * PER_W
        pltpu.sync_copy(idx_hbm.at[pl.ds(base, PER_W)], idx_v)
        sems = (sem0, sem1)

        def start_gather(g, b):
            pltpu.async_copy(
                table_hbm.at[idx_v.at[pl.ds(g * CHUNK, CHUNK)]],
                rows_v.at[b],
                sems[b],
            )

        def wait_gather(g, b):
            pltpu.make_async_copy(
                table_hbm.at[idx_v.at[pl.ds(g * CHUNK, CHUNK)]],
                rows_v.at[b],
                sems[b],
            ).wait()

        def step(g, b):
            wait_gather(g, b)

            @pl.when(g + 1 < NCHUNK)
            def _():
                start_gather(g + 1, 1 - b)

            pltpu.sync_copy(
                rows_v.at[b], out_hbm.at[pl.ds(base + g * CHUNK, CHUNK)]
            )

        start_gather(0, 0)

        def outer(p, carry):
            for b in range(2):
                step(2 * p + b, b)
            return carry

        lax.fori_loop(0, NCHUNK // 2, outer, 0)
        step(NCHUNK - 1, 0)

    return k


_sc_gather = _make_sc_gather()


def kernel(x, table):
    xf = x.reshape(-1).astype(jnp.int32)
    out = _sc_gather(table, xf)
    return out.reshape(BATCH, SEQ, D)


# speedup vs baseline: 1.2399x; 1.0019x over previous
import functools

import jax
import jax.numpy as jnp
from jax import lax
from jax.experimental import pallas as pl
from jax.experimental.pallas import tpu as pltpu
from jax.experimental.pallas import tpu_sc as plsc

VOCAB = 1000
BATCH = 1024
SEQ = 50
N = BATCH * SEQ
D = VOCAB
NUM_CORES = 2
NUM_SUBCORES = 16
NW = NUM_CORES * NUM_SUBCORES
PER_W = N // NW
CHUNK = 40
NCHUNK = PER_W // CHUNK


def _make_sc_gather():
    mesh = plsc.VectorSubcoreMesh(core_axis_name="c", subcore_axis_name="s")

    @functools.partial(
        pl.kernel,
        mesh=mesh,
        compiler_params=pltpu.CompilerParams(use_tc_tiling_on_sc=False),
        out_type=jax.ShapeDtypeStruct((N, D), jnp.float32),
        scratch_types=[
            pltpu.VMEM((PER_W,), jnp.int32),
            pltpu.VMEM((2, CHUNK, D), jnp.float32),
            pltpu.SemaphoreType.DMA,
            pltpu.SemaphoreType.DMA,
        ],
    )
    def k(table_hbm, idx_hbm, out_hbm, idx_v, rows_v, sem0, sem1):
        cid = lax.axis_index("c")
        sid = lax.axis_index("s")
        wid = sid * NUM_CORES + cid
        base = wid * PER_W
        pltpu.sync_copy(idx_hbm.at[pl.ds(base, PER_W)], idx_v)
        sems = (sem0, sem1)

        def start_gather(g, b):
            pltpu.async_copy(
                table_hbm.at[idx_v.at[pl.ds(g * CHUNK, CHUNK)]],
                rows_v.at[b],
                sems[b],
            )

        def wait_gather(g, b):
            pltpu.make_async_copy(
                table_hbm.at[idx_v.at[pl.ds(g * CHUNK, CHUNK)]],
                rows_v.at[b],
                sems[b],
            ).wait()

        def step(g, b):
            wait_gather(g, b)

            @pl.when(g + 1 < NCHUNK)
            def _():
                start_gather(g + 1, 1 - b)

            pltpu.sync_copy(
                rows_v.at[b], out_hbm.at[pl.ds(base + g * CHUNK, CHUNK)]
            )

        start_gather(0, 0)

        def outer(p, carry):
            for b in range(2):
                step(2 * p + b, b)
            return carry

        lax.fori_loop(0, NCHUNK // 2, outer, 0)
        if NCHUNK % 2:
            step(NCHUNK - 1, 0)

    return k


_sc_gather = _make_sc_gather()


def kernel(x, table):
    xf = x.reshape(-1).astype(jnp.int32)
    out = _sc_gather(table, xf)
    return out.reshape(BATCH, SEQ, D)
